# baseline (speedup 1.0000x reference)
import functools

import jax
import jax.numpy as jnp
from jax import lax
from jax.experimental import pallas as pl
from jax.experimental.pallas import tpu as pltpu
from jax.experimental.pallas import tpu_sc as plsc

D = 128
B = 256
T = 32
NLAYERS = 2
ROWS_PER_TENSOR = B * T * 3
TENSORS_PER_HALF = 3
HROWS = TENSORS_PER_HALF * ROWS_PER_TENSOR

NW = 32
CHUNK = 128
T_CHUNKS = ROWS_PER_TENSOR // (NW * CHUNK)
NODE_PER_WORKER = B // NW

RB = 3072
BATCHES_PER_BLOCK = RB // (T * 3)
H_BLOCKS = HROWS // RB


def _sc_gather_tensor_body(with_node, table, idx, nidx, out, nout, idx_v,
                           nidx_v, rows_a, rows_b, nrows_v, sem_a, sem_b):
    wid = lax.axis_index("s") * 2 + lax.axis_index("c")
    pltpu.sync_copy(idx.at[wid], idx_v)

    def store(c, rows):
        ch = wid * T_CHUNKS + c
        pltpu.sync_copy(rows, out.at[ch // 64, ch % 64])

    pltpu.async_copy(table.at[idx_v.at[0]], rows_a, sem_a)

    def pair(i, _):
        c = 2 * i
        pltpu.async_copy(table.at[idx_v.at[c + 1]], rows_b, sem_b)
        pltpu.make_async_copy(table.at[idx_v.at[c]], rows_a, sem_a).wait()
        store(c, rows_a)

        @pl.when(c + 2 < T_CHUNKS)
        def _():
            pltpu.async_copy(table.at[idx_v.at[c + 2]], rows_a, sem_a)

        pltpu.make_async_copy(table.at[idx_v.at[c + 1]], rows_b, sem_b).wait()
        store(c + 1, rows_b)
        return 0

    lax.fori_loop(0, T_CHUNKS // 2, pair, 0)

    if with_node:
        pltpu.sync_copy(nidx.at[wid], nidx_v)
        pltpu.async_copy(table.at[nidx_v], nrows_v, sem_a).wait()
        pltpu.sync_copy(nrows_v, nout.at[wid])


@functools.cache
def _get_sc_gather(with_node):
    out_type = [jax.ShapeDtypeStruct(
        (3, ROWS_PER_TENSOR // (3 * CHUNK), CHUNK, D), jnp.float32)]
    scratch = [
        pltpu.VMEM((T_CHUNKS, CHUNK), jnp.int32),
        pltpu.VMEM((NODE_PER_WORKER,), jnp.int32),
        pltpu.VMEM((CHUNK, D), jnp.float32),
        pltpu.VMEM((CHUNK, D), jnp.float32),
        pltpu.VMEM((NODE_PER_WORKER, D), jnp.float32),
        pltpu.SemaphoreType.DMA,
        pltpu.SemaphoreType.DMA,
    ]
    if with_node:
        out_type.append(jax.ShapeDtypeStruct((NW, NODE_PER_WORKER, D), jnp.float32))

        def body(table, idx, nidx, out, nout, idx_v, nidx_v, ra, rb, nv, sa, sb):
            _sc_gather_tensor_body(True, table, idx, nidx, out, nout, idx_v,
                                   nidx_v, ra, rb, nv, sa, sb)
    else:
        def body(table, idx, out, idx_v, nidx_v, ra, rb, nv, sa, sb):
            _sc_gather_tensor_body(False, table, idx, None, out, None, idx_v,
                                   nidx_v, ra, rb, nv, sa, sb)

    return pl.kernel(
        body,
        out_type=tuple(out_type),
        mesh=plsc.VectorSubcoreMesh(core_axis_name="c", subcore_axis_name="s"),
        scratch_types=scratch,
        compiler_params=pltpu.CompilerParams(use_tc_tiling_on_sc=True),
    )


def _graphlet_relu(x_ref, ws_ref, wm_ref, b_ref):
    g = RB // 3
    x3 = x_ref[...].reshape(3, g, D)
    ws = ws_ref[...]
    wm = wm_ref[...]
    b = b_ref[...]
    s = x3[0] + x3[1] + x3[2]
    t = lax.dot(s, wm, preferred_element_type=jnp.float32) + b
    xf = x3.reshape(RB, D)
    tb = jnp.broadcast_to(t[None], (3, g, D)).reshape(RB, D)
    z = lax.dot(xf, ws - wm, preferred_element_type=jnp.float32) + tb
    return jnp.maximum(z, 0.0)


def _mp_plain_body(x_ref, ws_ref, wm_ref, b_ref, m_ref):
    g = RB // 3
    y = _graphlet_relu(x_ref, ws_ref, wm_ref, b_ref).reshape(3, g, D)
    part = y[0] + y[1] + y[2]
    num = part.reshape(BATCHES_PER_BLOCK, T, D).sum(axis=1)
    m_ref[...] = (num * (1.0 / (T * 3 + 1e-6)))[None]


def _mp_masked_body(x_ref, ws_ref, wm_ref, b_ref, wmat_ref, m_ref):
    y = _graphlet_relu(x_ref, ws_ref, wm_ref, b_ref)
    wmat = wmat_ref[0]
    num = lax.dot(wmat, y, preferred_element_type=jnp.float32)
    den = jnp.sum(wmat, axis=1, keepdims=True)
    m_ref[...] = (num / (den + 1e-6))[None]


_BLKS = RB // 3 // CHUNK


def _mp_plain(rows, ws, wm, b):
    return pl.pallas_call(
        _mp_plain_body,
        grid=(8,),
        in_specs=[
            pl.BlockSpec((3, _BLKS, CHUNK, D), lambda i: (0, i, 0, 0)),
            pl.BlockSpec((D, D), lambda i: (0, 0)),
            pl.BlockSpec((D, D), lambda i: (0, 0)),
            pl.BlockSpec((1, D), lambda i: (0, 0)),
        ],
        out_specs=pl.BlockSpec((1, BATCHES_PER_BLOCK, D), lambda i: (i, 0, 0)),
        out_shape=jax.ShapeDtypeStruct((8, BATCHES_PER_BLOCK, D), jnp.float32),
    )(rows, ws, wm, b)


def _mp_masked(rows, ws, wm, b, wmat):
    return pl.pallas_call(
        _mp_masked_body,
        grid=(8,),
        in_specs=[
            pl.BlockSpec((3, _BLKS, CHUNK, D), lambda i: (0, i, 0, 0)),
            pl.BlockSpec((D, D), lambda i: (0, 0)),
            pl.BlockSpec((D, D), lambda i: (0, 0)),
            pl.BlockSpec((1, D), lambda i: (0, 0)),
            pl.BlockSpec((1, BATCHES_PER_BLOCK, RB), lambda i: (i, 0, 0)),
        ],
        out_specs=pl.BlockSpec((1, BATCHES_PER_BLOCK, D), lambda i: (i, 0, 0)),
        out_shape=jax.ShapeDtypeStruct((8, BATCHES_PER_BLOCK, D), jnp.float32),
    )(rows, ws, wm, b, wmat)


def _combine_body(t0_ref, nt0_ref, tn0_ref, t1_ref, nt1_ref, tn1_ref, ne_ref,
                  wg_ref, bg_ref, w1_ref, b1_ref, w2_ref, b2_ref, o_ref):
    def agg(m_ref, k):
        return (lax.dot(m_ref[...].reshape(B, D), wg_ref[k],
                        preferred_element_type=jnp.float32) + bg_ref[k])

    t_mean = 0.25 * (agg(t0_ref, 0) - agg(nt0_ref, 1)
                     + agg(t1_ref, 3) - agg(nt1_ref, 4))
    n_mean = 0.5 * (agg(tn0_ref, 2) + agg(tn1_ref, 5))
    h = (lax.dot(ne_ref[...].reshape(B, D), w1_ref[0],
                 preferred_element_type=jnp.float32)
         + lax.dot(t_mean, w1_ref[1], preferred_element_type=jnp.float32)
         + lax.dot(n_mean, w1_ref[2], preferred_element_type=jnp.float32)
         + b1_ref[...])
    h1 = jnp.maximum(h, 0.0)
    o_ref[...] = jnp.sum(h1 * w2_ref[...], axis=1, keepdims=True) + b2_ref[...]


def _combine(ms, node_e, wg_all, bg_all, w1, b1, w2, b2):
    return pl.pallas_call(
        _combine_body,
        out_shape=jax.ShapeDtypeStruct((B, 1), jnp.float32),
    )(*ms, node_e, wg_all, bg_all, w1, b1, w2, b2)


def kernel(params, node_id, trianglelogic, squarelogic, triangle, notriangle,
           square, nosquare, triangle_neighbor, triangle_mask, square_neighbor,
           square_mask):
    emb = params['embedding']
    eye = jnp.eye(BATCHES_PER_BLOCK, dtype=jnp.float32)

    def node_major_idx(a):
        return (jnp.transpose(a, (2, 0, 1)).reshape(-1)
                .astype(jnp.int32).reshape(NW, T_CHUNKS, CHUNK))

    def build_wmat(mask_l):
        mt = jnp.transpose(mask_l, (2, 0, 1)).reshape(3, 8, BATCHES_PER_BLOCK, T)
        wm5 = (mt.transpose(1, 0, 2, 3)[:, None]
               * eye[None, :, None, :, None])
        return wm5.reshape(8, BATCHES_PER_BLOCK, RB)

    wg_l, bg_l = [], []
    for l in range(NLAYERS):
        for name in ('tpa', 'tga', 'tna'):
            wg, bg = params[name][l][0]
            wg_l.append(wg)
            bg_l.append(bg)
    nidx = node_id.astype(jnp.int32).reshape(NW, NODE_PER_WORKER)
    wg_all = jnp.stack(wg_l)
    bg_all = jnp.stack(bg_l).reshape(2 * TENSORS_PER_HALF, 1, D)
    (w1, b1), (w2, b2) = params['combine']
    w1r = w1.reshape(3, D, D)
    b1r = b1.reshape(1, D)
    w2r = w2.reshape(1, D)
    b2r = b2.reshape(1, 1)

    ms = []
    node_e = None
    for l in range(NLAYERS):
        (ws_t, wm_t, bias_t) = params['tmp'][l][0]
        (ws_n, wm_n, bias_n) = params['tnp'][l][0]
        wmat = build_wmat(triangle_mask[:, l])
        for src, ws, wm, bias, masked in (
                (triangle, ws_t, wm_t, bias_t, False),
                (notriangle, ws_t, wm_t, bias_t, False),
                (triangle_neighbor, ws_n, wm_n, bias_n, True)):
            idx = node_major_idx(src[:, l])
            if node_e is None:
                rows, node_e = _get_sc_gather(True)(emb, idx, nidx)
            else:
                rows, = _get_sc_gather(False)(emb, idx)
            br = bias.reshape(1, D)
            if masked:
                ms.append(_mp_masked(rows, ws, wm, br, wmat))
            else:
                ms.append(_mp_plain(rows, ws, wm, br))
    return _combine(ms, node_e, wg_all, bg_all, w1r, b1r, w2r, b2r)


# speedup vs baseline: 1.0405x; 1.0405x over previous
import functools

import jax
import jax.numpy as jnp
from jax import lax
from jax.experimental import pallas as pl
from jax.experimental.pallas import tpu as pltpu
from jax.experimental.pallas import tpu_sc as plsc

D = 128
B = 256
T = 32
NLAYERS = 2
ROWS_PER_TENSOR = B * T * 3
TENSORS_PER_HALF = 3
HROWS = TENSORS_PER_HALF * ROWS_PER_TENSOR

NW = 32
CHUNK = 128
T_CHUNKS = ROWS_PER_TENSOR // (NW * CHUNK)
NODE_PER_WORKER = B // NW

RB = 3072
BATCHES_PER_BLOCK = RB // (T * 3)
H_BLOCKS = HROWS // RB


def _sc_gather_tensor_body(with_node, table, idx, nidx, out, nout, idx_v,
                           nidx_v, bufs, nrows_v, gsems, ssem):
    wid = lax.axis_index("s") * 2 + lax.axis_index("c")
    pltpu.sync_copy(idx.at[wid], idx_v)

    for c in range(T_CHUNKS):
        pltpu.async_copy(table.at[idx_v.at[c]], bufs[c], gsems[c])
    for c in range(T_CHUNKS):
        ch = wid * T_CHUNKS + c
        pltpu.make_async_copy(table.at[idx_v.at[c]], bufs[c], gsems[c]).wait()
        pltpu.async_copy(bufs[c], out.at[ch // 64, ch % 64], ssem)
    for c in range(T_CHUNKS):
        pltpu.make_async_copy(bufs[c], out.at[0, 0], ssem).wait()

    if with_node:
        pltpu.sync_copy(nidx.at[wid], nidx_v)
        pltpu.async_copy(table.at[nidx_v], nrows_v, gsems[0]).wait()
        pltpu.sync_copy(nrows_v, nout.at[wid])


@functools.cache
def _get_sc_gather(with_node):
    out_type = [jax.ShapeDtypeStruct(
        (3, ROWS_PER_TENSOR // (3 * CHUNK), CHUNK, D), jnp.float32)]
    scratch = [
        pltpu.VMEM((T_CHUNKS, CHUNK), jnp.int32),
        pltpu.VMEM((NODE_PER_WORKER,), jnp.int32),
        [pltpu.VMEM((CHUNK, D), jnp.float32)] * T_CHUNKS,
        pltpu.VMEM((NODE_PER_WORKER, D), jnp.float32),
        [pltpu.SemaphoreType.DMA] * T_CHUNKS,
        pltpu.SemaphoreType.DMA,
    ]
    if with_node:
        out_type.append(jax.ShapeDtypeStruct((NW, NODE_PER_WORKER, D), jnp.float32))

        def body(table, idx, nidx, out, nout, idx_v, nidx_v, bufs, nv, gs, ss):
            _sc_gather_tensor_body(True, table, idx, nidx, out, nout, idx_v,
                                   nidx_v, bufs, nv, gs, ss)
    else:
        def body(table, idx, out, idx_v, nidx_v, bufs, nv, gs, ss):
            _sc_gather_tensor_body(False, table, idx, None, out, None, idx_v,
                                   nidx_v, bufs, nv, gs, ss)

    return pl.kernel(
        body,
        out_type=tuple(out_type),
        mesh=plsc.VectorSubcoreMesh(core_axis_name="c", subcore_axis_name="s"),
        scratch_types=scratch,
        compiler_params=pltpu.CompilerParams(use_tc_tiling_on_sc=True),
    )


def _graphlet_relu(x_ref, ws_ref, wm_ref, b_ref):
    g = RB // 3
    x3 = x_ref[...].reshape(3, g, D)
    ws = ws_ref[...]
    wm = wm_ref[...]
    b = b_ref[...]
    s = x3[0] + x3[1] + x3[2]
    t = lax.dot(s, wm, preferred_element_type=jnp.float32) + b
    xf = x3.reshape(RB, D)
    tb = jnp.broadcast_to(t[None], (3, g, D)).reshape(RB, D)
    z = lax.dot(xf, ws - wm, preferred_element_type=jnp.float32) + tb
    return jnp.maximum(z, 0.0)


def _mp_plain_body(x_ref, ws_ref, wm_ref, b_ref, m_ref):
    g = RB // 3
    y = _graphlet_relu(x_ref, ws_ref, wm_ref, b_ref).reshape(3, g, D)
    part = y[0] + y[1] + y[2]
    num = part.reshape(BATCHES_PER_BLOCK, T, D).sum(axis=1)
    m_ref[...] = (num * (1.0 / (T * 3 + 1e-6)))[None]


def _mp_masked_body(x_ref, ws_ref, wm_ref, b_ref, wmat_ref, m_ref):
    y = _graphlet_relu(x_ref, ws_ref, wm_ref, b_ref)
    wmat = wmat_ref[0]
    num = lax.dot(wmat, y, preferred_element_type=jnp.float32)
    den = jnp.sum(wmat, axis=1, keepdims=True)
    m_ref[...] = (num / (den + 1e-6))[None]


_BLKS = RB // 3 // CHUNK


def _mp_plain(rows, ws, wm, b):
    return pl.pallas_call(
        _mp_plain_body,
        grid=(8,),
        in_specs=[
            pl.BlockSpec((3, _BLKS, CHUNK, D), lambda i: (0, i, 0, 0)),
            pl.BlockSpec((D, D), lambda i: (0, 0)),
            pl.BlockSpec((D, D), lambda i: (0, 0)),
            pl.BlockSpec((1, D), lambda i: (0, 0)),
        ],
        out_specs=pl.BlockSpec((1, BATCHES_PER_BLOCK, D), lambda i: (i, 0, 0)),
        out_shape=jax.ShapeDtypeStruct((8, BATCHES_PER_BLOCK, D), jnp.float32),
    )(rows, ws, wm, b)


def _mp_masked(rows, ws, wm, b, wmat):
    return pl.pallas_call(
        _mp_masked_body,
        grid=(8,),
        in_specs=[
            pl.BlockSpec((3, _BLKS, CHUNK, D), lambda i: (0, i, 0, 0)),
            pl.BlockSpec((D, D), lambda i: (0, 0)),
            pl.BlockSpec((D, D), lambda i: (0, 0)),
            pl.BlockSpec((1, D), lambda i: (0, 0)),
            pl.BlockSpec((1, BATCHES_PER_BLOCK, RB), lambda i: (i, 0, 0)),
        ],
        out_specs=pl.BlockSpec((1, BATCHES_PER_BLOCK, D), lambda i: (i, 0, 0)),
        out_shape=jax.ShapeDtypeStruct((8, BATCHES_PER_BLOCK, D), jnp.float32),
    )(rows, ws, wm, b, wmat)


def _combine_body(t0_ref, nt0_ref, tn0_ref, t1_ref, nt1_ref, tn1_ref, ne_ref,
                  wg_ref, bg_ref, w1_ref, b1_ref, w2_ref, b2_ref, o_ref):
    def agg(m_ref, k):
        return (lax.dot(m_ref[...].reshape(B, D), wg_ref[k],
                        preferred_element_type=jnp.float32) + bg_ref[k])

    t_mean = 0.25 * (agg(t0_ref, 0) - agg(nt0_ref, 1)
                     + agg(t1_ref, 3) - agg(nt1_ref, 4))
    n_mean = 0.5 * (agg(tn0_ref, 2) + agg(tn1_ref, 5))
    h = (lax.dot(ne_ref[...].reshape(B, D), w1_ref[0],
                 preferred_element_type=jnp.float32)
         + lax.dot(t_mean, w1_ref[1], preferred_element_type=jnp.float32)
         + lax.dot(n_mean, w1_ref[2], preferred_element_type=jnp.float32)
         + b1_ref[...])
    h1 = jnp.maximum(h, 0.0)
    o_ref[...] = jnp.sum(h1 * w2_ref[...], axis=1, keepdims=True) + b2_ref[...]


def _combine(ms, node_e, wg_all, bg_all, w1, b1, w2, b2):
    return pl.pallas_call(
        _combine_body,
        out_shape=jax.ShapeDtypeStruct((B, 1), jnp.float32),
    )(*ms, node_e, wg_all, bg_all, w1, b1, w2, b2)


def kernel(params, node_id, trianglelogic, squarelogic, triangle, notriangle,
           square, nosquare, triangle_neighbor, triangle_mask, square_neighbor,
           square_mask):
    emb = params['embedding']
    eye = jnp.eye(BATCHES_PER_BLOCK, dtype=jnp.float32)

    def node_major_idx(a):
        return (jnp.transpose(a, (2, 0, 1)).reshape(-1)
                .astype(jnp.int32).reshape(NW, T_CHUNKS, CHUNK))

    def build_wmat(mask_l):
        mt = jnp.transpose(mask_l, (2, 0, 1)).reshape(3, 8, BATCHES_PER_BLOCK, T)
        wm5 = (mt.transpose(1, 0, 2, 3)[:, None]
               * eye[None, :, None, :, None])
        return wm5.reshape(8, BATCHES_PER_BLOCK, RB)

    wg_l, bg_l = [], []
    for l in range(NLAYERS):
        for name in ('tpa', 'tga', 'tna'):
            wg, bg = params[name][l][0]
            wg_l.append(wg)
            bg_l.append(bg)
    nidx = node_id.astype(jnp.int32).reshape(NW, NODE_PER_WORKER)
    wg_all = jnp.stack(wg_l)
    bg_all = jnp.stack(bg_l).reshape(2 * TENSORS_PER_HALF, 1, D)
    (w1, b1), (w2, b2) = params['combine']
    w1r = w1.reshape(3, D, D)
    b1r = b1.reshape(1, D)
    w2r = w2.reshape(1, D)
    b2r = b2.reshape(1, 1)

    ms = []
    node_e = None
    for l in range(NLAYERS):
        (ws_t, wm_t, bias_t) = params['tmp'][l][0]
        (ws_n, wm_n, bias_n) = params['tnp'][l][0]
        wmat = build_wmat(triangle_mask[:, l])
        for src, ws, wm, bias, masked in (
                (triangle, ws_t, wm_t, bias_t, False),
                (notriangle, ws_t, wm_t, bias_t, False),
                (triangle_neighbor, ws_n, wm_n, bias_n, True)):
            idx = node_major_idx(src[:, l])
            if node_e is None:
                rows, node_e = _get_sc_gather(True)(emb, idx, nidx)
            else:
                rows, = _get_sc_gather(False)(emb, idx)
            br = bias.reshape(1, D)
            if masked:
                ms.append(_mp_masked(rows, ws, wm, br, wmat))
            else:
                ms.append(_mp_plain(rows, ws, wm, br))
    return _combine(ms, node_e, wg_all, bg_all, w1r, b1r, w2r, b2r)
